# Initial kernel scaffold; baseline (speedup 1.0000x reference)
import jax
import jax.numpy as jnp
from jax.experimental import pallas as pl


def kernel(x, edge_index, edge_attr, batch, W1a, V1a, W2a, V2a, Wm1, bm1, Wm2, bm2, We1, Ve1, We2, Ve2, Wh1, bh1, Wh2, bh2, Wf, bf):
    raise NotImplementedError("write your pallas kernel here")


# speedup vs baseline: 3.0862x; 3.0862x over previous
import functools

import jax
import jax.numpy as jnp
import numpy as np
from jax import lax
from jax.experimental import pallas as pl
from jax.experimental.pallas import tpu as pltpu
from jax.experimental.pallas import tpu_sc as plsc

N = 10000
E = 320000
D = 128
G = 128
K = E // 2

NC = 2
NS = 16
NW = NC * NS
EPW = E // NW
CE = 80
NCHUNK = EPW // CE
RPT = 624
RB = 1000
HI = 512
MIN32 = np.int32(-2147483648)
MIN32XF = np.int32(-65536)

_mesh = plsc.VectorSubcoreMesh(core_axis_name="c", subcore_axis_name="s")


def _bf16r(x):
    u = lax.bitcast_convert_type(x, jnp.int32)
    r = (u + jnp.int32(0x7FFF) + ((u >> 16) & 1)) & MIN32XF
    return lax.bitcast_convert_type(r, jnp.float32)


def _acc_prologue(z_h, acc_sh, sid):
    pltpu.sync_copy(z_h.at[pl.ds(0, RPT)], acc_sh.at[pl.ds(sid * RPT, RPT)])

    @pl.when(sid == NS - 1)
    def _():
        pltpu.sync_copy(z_h.at[pl.ds(0, N - NS * RPT)],
                        acc_sh.at[pl.ds(NS * RPT, N - NS * RPT)])

    plsc.subcore_barrier()


def _acc_epilogue(acc_sh, out0_h, out1_h, cid, sid):
    plsc.subcore_barrier()
    rs = pl.ds(sid * RPT, RPT)
    tail = pl.ds(NS * RPT, N - NS * RPT)

    @pl.when(cid == 0)
    def _():
        pltpu.sync_copy(acc_sh.at[rs], out0_h.at[rs])

        @pl.when(sid == NS - 1)
        def _():
            pltpu.sync_copy(acc_sh.at[tail], out0_h.at[tail])

    @pl.when(cid == 1)
    def _():
        pltpu.sync_copy(acc_sh.at[rs], out1_h.at[rs])

        @pl.when(sid == NS - 1)
        def _():
            pltpu.sync_copy(acc_sh.at[tail], out1_h.at[tail])


@functools.partial(
    pl.kernel,
    out_type=(jax.ShapeDtypeStruct((N, D), jnp.float32),
              jax.ShapeDtypeStruct((N, D), jnp.float32)),
    mesh=_mesh,
    scratch_types=[
        pltpu.VMEM((CE,), jnp.int32),
        pltpu.VMEM((CE,), jnp.int32),
        pltpu.VMEM((CE,), jnp.float32),
        pltpu.VMEM((CE, D), jnp.float32),
        pltpu.VMEM_SHARED((N, D), jnp.float32),
        pltpu.SemaphoreType.DMA,
    ],
)
def _sc_conv(table_h, src_h, dst_h, w_h, z_h, out0_h, out1_h,
             src_v, dst_v, w_v, rows_v, acc_sh, sem):
    cid = lax.axis_index("c")
    sid = lax.axis_index("s")
    wid = sid * NC + cid
    _acc_prologue(z_h, acc_sh, sid)

    def chunk_body(ci, _):
        base = wid * EPW + ci * CE
        pltpu.sync_copy(src_h.at[pl.ds(base, CE)], src_v)
        pltpu.sync_copy(dst_h.at[pl.ds(base, CE)], dst_v)
        pltpu.sync_copy(w_h.at[pl.ds(base, CE)], w_v)
        pltpu.async_copy(table_h.at[src_v], rows_v, sem).wait()

        def group_body(g, _):
            wv = w_v[pl.ds(g * 16, 16)]
            for l in range(16):
                sv = jnp.full((16,), wv[l], jnp.float32)
                j = g * 16 + l
                for d in range(D // 16):
                    sl = pl.ds(d * 16, 16)
                    rows_v[j, sl] = rows_v[j, sl] * sv
            return 0

        lax.fori_loop(0, CE // 16, group_body, 0)
        pltpu.sync_copy(rows_v, acc_sh.at[dst_v], add=True)
        return 0

    lax.fori_loop(0, NCHUNK, chunk_body, 0)
    _acc_epilogue(acc_sh, out0_h, out1_h, cid, sid)


@functools.partial(
    pl.kernel,
    out_type=(jax.ShapeDtypeStruct((N, D), jnp.float32),
              jax.ShapeDtypeStruct((N, D), jnp.float32)),
    mesh=_mesh,
    scratch_types=[
        pltpu.VMEM((CE,), jnp.int32),
        pltpu.VMEM((CE,), jnp.float32),
        pltpu.VMEM((CE, D), jnp.float32),
        pltpu.VMEM_SHARED((N, D), jnp.float32),
        pltpu.SemaphoreType.DMA,
    ],
)
def _sc_deg(dst_h, w_h, z_h, out0_h, out1_h, dst_v, w_v, rows_v, acc_sh, sem):
    cid = lax.axis_index("c")
    sid = lax.axis_index("s")
    wid = sid * NC + cid
    _acc_prologue(z_h, acc_sh, sid)

    def chunk_body(ci, _):
        base = wid * EPW + ci * CE
        pltpu.sync_copy(dst_h.at[pl.ds(base, CE)], dst_v)
        pltpu.sync_copy(w_h.at[pl.ds(base, CE)], w_v)

        def group_body(g, _):
            wv = w_v[pl.ds(g * 16, 16)]
            for l in range(16):
                sv = jnp.full((16,), wv[l], jnp.float32)
                j = g * 16 + l
                for d in range(D // 16):
                    rows_v[j, pl.ds(d * 16, 16)] = sv
            return 0

        lax.fori_loop(0, CE // 16, group_body, 0)
        pltpu.sync_copy(rows_v, acc_sh.at[dst_v], add=True)
        return 0

    lax.fori_loop(0, NCHUNK, chunk_body, 0)
    _acc_epilogue(acc_sh, out0_h, out1_h, cid, sid)


@functools.partial(
    pl.kernel,
    out_type=jax.ShapeDtypeStruct((E * 16,), jnp.float32),
    mesh=_mesh,
    scratch_types=[
        pltpu.VMEM((CE,), jnp.int32),
        pltpu.VMEM((CE,), jnp.int32),
        pltpu.VMEM((CE, HI), jnp.float32),
        pltpu.VMEM((CE, HI), jnp.float32),
        pltpu.VMEM((HI,), jnp.float32),
        pltpu.VMEM((CE * 16,), jnp.float32),
        pltpu.SemaphoreType.DMA,
        pltpu.SemaphoreType.DMA,
    ],
)
def _sc_score(a_h, b_h, src_h, dst_h, wm2_h, s16_h,
              src_v, dst_v, a_v, b_v, w2_v, s_v, semA, semB):
    cid = lax.axis_index("c")
    sid = lax.axis_index("s")
    wid = sid * NC + cid
    pltpu.sync_copy(wm2_h, w2_v)
    w2regs = [_bf16r(w2_v[pl.ds(d * 16, 16)]) for d in range(HI // 16)]

    def chunk_body(ci, _):
        base = wid * EPW + ci * CE
        pltpu.sync_copy(src_h.at[pl.ds(base, CE)], src_v)
        pltpu.sync_copy(dst_h.at[pl.ds(base, CE)], dst_v)
        ca = pltpu.async_copy(a_h.at[src_v], a_v, semA)
        cb = pltpu.async_copy(b_h.at[dst_v], b_v, semB)
        ca.wait()
        cb.wait()

        def edge_body(j, _):
            acc = jnp.zeros((16,), jnp.float32)
            for d in range(HI // 16):
                sl = pl.ds(d * 16, 16)
                hid = _bf16r(jnp.maximum(a_v[j, sl] + b_v[j, sl], 0.0))
                acc = acc + hid * w2regs[d]
            s_v[pl.ds(j * 16, 16)] = acc
            return 0

        lax.fori_loop(0, CE, edge_body, 0)
        pltpu.sync_copy(s_v, s16_h.at[pl.ds(base * 16, CE * 16)])
        return 0

    lax.fori_loop(0, NCHUNK, chunk_body, 0)


def _rowspec(w):
    return pl.BlockSpec((RB, w), lambda i: (i, 0))


def _full(shape):
    return pl.BlockSpec(shape, lambda i: tuple(0 for _ in shape))


def _tc_call(body, n_out, out_w, grid=N // RB):
    outs = tuple(jax.ShapeDtypeStruct((grid * RB, w), jnp.float32)
                 for w in out_w)
    return functools.partial(
        pl.pallas_call, body, grid=(grid,),
        out_shape=outs if n_out > 1 else outs[0],
        out_specs=tuple(_rowspec(w) for w in out_w) if n_out > 1
        else _rowspec(out_w[0]))


def _tca_body(d0, d1, x, w1, v1, dinv16, hs1, xv1):
    dinv = 1.0 / jnp.sqrt(jnp.maximum(d0[:, 0:1] + d1[:, 0:1], 1e-12))
    dinv16[...] = jnp.broadcast_to(dinv, (RB, 16))
    hs1[...] = jnp.dot(x[...], w1[...],
                       preferred_element_type=jnp.float32) * dinv
    xv1[...] = jnp.dot(x[...], v1[...], preferred_element_type=jnp.float32)


def _tcb_body(a0, a1, dinv16, xv1, w2, v2, hs2, xv2):
    dinv = dinv16[:, 0:1]
    x1 = jnp.maximum((a0[...] + a1[...]) * dinv + xv1[...], 0.0)
    hs2[...] = jnp.dot(x1, w2[...], preferred_element_type=jnp.float32) * dinv
    xv2[...] = jnp.dot(x1, v2[...], preferred_element_type=jnp.float32)


def _tcc_body(a0, a1, dinv16, xv2, wm1a, wm1b, bm1, x2o, ao, bo):
    x2 = (a0[...] + a1[...]) * dinv16[:, 0:1] + xv2[...]
    x2o[...] = x2
    ao[...] = jnp.dot(x2, wm1a[...], preferred_element_type=jnp.float32)
    bo[...] = jnp.dot(x2, wm1b[...], preferred_element_type=jnp.float32) + bm1[...]


def _tcd1_body(s16r, bm2, so):
    ii = lax.broadcasted_iota(jnp.int32, (128, 8), 0)
    jj = lax.broadcasted_iota(jnp.int32, (128, 8), 1)
    bd = ((ii // 16) == jj).astype(jnp.float32)
    so[...] = jnp.dot(s16r[...], bd, preferred_element_type=jnp.float32,
                      precision=lax.Precision.HIGHEST) + bm2[...]


def _tcd2_body(s2d, ew2d, wo):
    s = s2d[...]
    bits = lax.bitcast_convert_type(s, jnp.int32)
    key = jnp.where(bits >= 0, bits, bits ^ jnp.int32(0x7FFFFFFF))

    def bisect(i, P):
        cand = P | (jnp.int32(1) << (31 - i))
        c = jnp.sum((key >= (cand ^ MIN32)).astype(jnp.int32))
        return jnp.where(c >= K, cand, P)

    T = lax.fori_loop(0, 32, bisect, jnp.int32(0)) ^ MIN32
    gt = key > T
    need = jnp.int32(K) - jnp.sum(gt.astype(jnp.int32))
    eq = key == T
    sh = s.shape
    lin = (lax.broadcasted_iota(jnp.int32, sh, 0) * sh[1]
           + lax.broadcasted_iota(jnp.int32, sh, 1))

    def tie_body(c):
        last, rem = c
        m = jnp.min(jnp.where(eq & (lin > last), lin, jnp.int32(2147483647)))
        return m, rem - 1

    last, _ = lax.while_loop(lambda c: c[1] > 0, tie_body,
                             (jnp.int32(-1), need))
    sel = gt | (eq & (lin <= last))
    wo[...] = jnp.where(sel, ew2d[...] * jax.nn.sigmoid(s), 0.0)


def _tce_body(d0, d1, x2, we1, ve1, dinv16, hs3, xv3):
    dinv = 1.0 / jnp.sqrt(jnp.maximum(d0[:, 0:1] + d1[:, 0:1], 1e-12))
    dinv16[...] = jnp.broadcast_to(dinv, (RB, 16))
    hs3[...] = jnp.dot(x2[...], we1[...],
                       preferred_element_type=jnp.float32) * dinv
    xv3[...] = jnp.dot(x2[...], ve1[...], preferred_element_type=jnp.float32)


def _tcf_body(a0, a1, dinv16, xv3, we2, ve2, h4so, xv4o):
    dinv = dinv16[:, 0:1]
    node3 = jnp.maximum((a0[...] + a1[...]) * dinv + xv3[...], 0.0)
    h4so[...] = jnp.dot(node3, we2[...],
                        preferred_element_type=jnp.float32) * dinv
    xv4o[...] = jnp.dot(node3, ve2[...], preferred_element_type=jnp.float32)


def _tcg_body(z0, z1, dinv16, xv4, wf, batch, bf, out):
    node4 = (z0[...] + z1[...]) * dinv16[:, 0:1] + xv4[...]
    oh = (batch[...] == lax.broadcasted_iota(jnp.int32, (G, 1), 0)
          ).astype(jnp.float32)
    counts = jnp.sum(oh, axis=1, keepdims=True)
    pooled = jnp.dot(oh, node4, preferred_element_type=jnp.float32,
                     precision=lax.Precision.HIGHEST)
    gq = pooled / jnp.maximum(counts, 1.0)
    out[...] = jnp.dot(gq, wf[...],
                       preferred_element_type=jnp.float32) + bf[...]


def kernel(x, edge_index, edge_attr, batch, W1a, V1a, W2a, V2a, Wm1, bm1,
           Wm2, bm2, We1, Ve1, We2, Ve2, Wh1, bh1, Wh2, bh2, Wf, bf):
    f32 = jnp.float32
    src = edge_index[0]
    dst = edge_index[1]
    ew = edge_attr.reshape(E)
    z128 = jnp.zeros((RPT, D), f32)

    d1p0, d1p1 = _sc_deg(dst, ew, z128)

    tca = _tc_call(_tca_body, 3, (16, 128, 128))(
        in_specs=[_rowspec(128), _rowspec(128), _rowspec(128),
                  _full((128, 128)), _full((128, 128))])
    dinv116, hs1, xv1 = tca(d1p0, d1p1, x, W1a, V1a)

    a1p0, a1p1 = _sc_conv(hs1, src, dst, ew, z128)

    tcb = _tc_call(_tcb_body, 2, (128, 128))(
        in_specs=[_rowspec(128), _rowspec(128), _rowspec(16), _rowspec(128),
                  _full((128, 128)), _full((128, 128))])
    hs2, xv2 = tcb(a1p0, a1p1, dinv116, xv1, W2a, V2a)

    a2p0, a2p1 = _sc_conv(hs2, src, dst, ew, z128)

    tcc = _tc_call(_tcc_body, 3, (128, HI, HI))(
        in_specs=[_rowspec(128), _rowspec(128), _rowspec(16), _rowspec(128),
                  _full((128, HI)), _full((128, HI)), _full((1, HI))])
    x2, A, B = tcc(a2p0, a2p1, dinv116, xv2, Wm1[:D], Wm1[D:],
                   bm1.reshape(1, HI))

    s16f = _sc_score(A, B, src, dst, Wm2.reshape(HI))
    s16r = s16f.reshape(E // 8, 128)

    tcd1 = functools.partial(
        pl.pallas_call, _tcd1_body, grid=(E // 8 // 8000,),
        out_shape=jax.ShapeDtypeStruct((E // 8, 8), f32),
        out_specs=pl.BlockSpec((8000, 8), lambda i: (i, 0)))(
            in_specs=[pl.BlockSpec((8000, 128), lambda i: (i, 0)),
                      _full((1, 1))])
    s1 = tcd1(s16r, bm2.reshape(1, 1))
    s2d = s1.reshape(E // 128, 128)

    tcd2 = functools.partial(
        pl.pallas_call, _tcd2_body, grid=(1,),
        out_shape=jax.ShapeDtypeStruct((E // 128, 128), f32),
        out_specs=_full((E // 128, 128)))(
            in_specs=[_full((E // 128, 128)), _full((E // 128, 128))])
    w = tcd2(s2d, ew.reshape(E // 128, 128)).reshape(E)

    d3p0, d3p1 = _sc_deg(dst, w, z128)

    tce = _tc_call(_tce_body, 3, (16, 128, 128))(
        in_specs=[_rowspec(128), _rowspec(128), _rowspec(128),
                  _full((128, 128)), _full((128, 128))])
    dinv316, hs3, xv3 = tce(d3p0, d3p1, x2, We1, Ve1)

    a3p0, a3p1 = _sc_conv(hs3, src, dst, w, z128)

    tcf = _tc_call(_tcf_body, 2, (128, 128))(
        in_specs=[_rowspec(128), _rowspec(128), _rowspec(16), _rowspec(128),
                  _full((128, 128)), _full((128, 128))])
    h4s, xv4 = tcf(a3p0, a3p1, dinv316, xv3, We2, Ve2)

    azp0, azp1 = _sc_conv(h4s, src, dst, w, z128)

    tcg = functools.partial(
        pl.pallas_call, _tcg_body, grid=(1,),
        out_shape=jax.ShapeDtypeStruct((G, 2), f32),
        out_specs=_full((G, 2)))(
            in_specs=[pl.BlockSpec((N, 128), lambda i: (0, 0)),
                      pl.BlockSpec((N, 128), lambda i: (0, 0)),
                      pl.BlockSpec((N, 16), lambda i: (0, 0)),
                      pl.BlockSpec((N, 128), lambda i: (0, 0)),
                      _full((128, 2)),
                      pl.BlockSpec((1, N), lambda i: (0, 0)),
                      _full((1, 2))])
    return tcg(azp0, azp1, dinv316, xv4, Wf,
               batch.reshape(1, N).astype(jnp.int32), bf.reshape(1, 2))


# speedup vs baseline: 3.5646x; 1.1550x over previous
import functools

import jax
import jax.numpy as jnp
import numpy as np
from jax import lax
from jax.experimental import pallas as pl
from jax.experimental.pallas import tpu as pltpu
from jax.experimental.pallas import tpu_sc as plsc

N = 10000
E = 320000
D = 128
G = 128
K = E // 2

NC = 2
NS = 16
NW = NC * NS
EPW = E // NW
CE = 80
NCHUNK = EPW // CE
RPT = 624
RB = 1000
HI = 512
MIN32 = np.int32(-2147483648)
MIN32XF = np.int32(-65536)

_mesh = plsc.VectorSubcoreMesh(core_axis_name="c", subcore_axis_name="s")


def _bf16r(x):
    u = lax.bitcast_convert_type(x, jnp.int32)
    r = (u + jnp.int32(0x7FFF) + ((u >> 16) & 1)) & MIN32XF
    return lax.bitcast_convert_type(r, jnp.float32)


def _acc_prologue(z_h, acc_sh, sid):
    pltpu.sync_copy(z_h.at[pl.ds(0, RPT)], acc_sh.at[pl.ds(sid * RPT, RPT)])

    @pl.when(sid == NS - 1)
    def _():
        pltpu.sync_copy(z_h.at[pl.ds(0, N - NS * RPT)],
                        acc_sh.at[pl.ds(NS * RPT, N - NS * RPT)])

    plsc.subcore_barrier()


def _acc_epilogue(acc_sh, out0_h, out1_h, cid, sid):
    plsc.subcore_barrier()
    rs = pl.ds(sid * RPT, RPT)
    tail = pl.ds(NS * RPT, N - NS * RPT)

    @pl.when(cid == 0)
    def _():
        pltpu.sync_copy(acc_sh.at[rs], out0_h.at[rs])

        @pl.when(sid == NS - 1)
        def _():
            pltpu.sync_copy(acc_sh.at[tail], out0_h.at[tail])

    @pl.when(cid == 1)
    def _():
        pltpu.sync_copy(acc_sh.at[rs], out1_h.at[rs])

        @pl.when(sid == NS - 1)
        def _():
            pltpu.sync_copy(acc_sh.at[tail], out1_h.at[tail])


CEC = 40
NCHC = EPW // CEC
NPC = NCHC // 2


@functools.partial(
    pl.kernel,
    out_type=(jax.ShapeDtypeStruct((N, D), jnp.float32),
              jax.ShapeDtypeStruct((N, D), jnp.float32)),
    mesh=_mesh,
    scratch_types=(
        [pltpu.VMEM((CEC,), jnp.int32)] * 4
        + [pltpu.VMEM((CEC,), jnp.float32)] * 2
        + [pltpu.VMEM((CEC, D), jnp.float32)] * 4
        + [pltpu.VMEM_SHARED((N, D), jnp.float32)]
        + [pltpu.SemaphoreType.DMA] * 4
    ),
)
def _sc_conv(table_h, src_h, dst_h, w_h, z_h, out0_h, out1_h,
             src0, src1, dst0, dst1, w0, w1, rows0, rows1, msg0, msg1,
             acc_sh, semG0, semG1, semSc0, semSc1):
    cid = lax.axis_index("c")
    sid = lax.axis_index("s")
    wid = sid * NC + cid
    ebase = wid * EPW
    srcb, dstb, wb = (src0, src1), (dst0, dst1), (w0, w1)
    rowsb, msgb = (rows0, rows1), (msg0, msg1)
    semG, semSc = (semG0, semG1), (semSc0, semSc1)
    _acc_prologue(z_h, acc_sh, sid)

    def load_idx(g, nb):
        base = ebase + g * CEC
        pltpu.sync_copy(src_h.at[pl.ds(base, CEC)], srcb[nb])
        pltpu.sync_copy(dst_h.at[pl.ds(base, CEC)], dstb[nb])
        pltpu.sync_copy(w_h.at[pl.ds(base, CEC)], wb[nb])

    load_idx(0, 0)
    pltpu.async_copy(table_h.at[srcb[0]], rowsb[0], semG[0])

    def scale_rows(b):
        def group_body(g16, _):
            wv = wb[b][pl.ds(g16 * 16, 16)]
            for l in range(16):
                sv = jnp.full((16,), wv[l], jnp.float32)
                j = g16 * 16 + l
                for d in range(D // 16):
                    sl = pl.ds(d * 16, 16)
                    msgb[b][j, sl] = rowsb[b][j, sl] * sv
            return 0

        lax.fori_loop(0, CEC // 16, group_body, 0)
        wv = wb[b][pl.ds(CEC - 16, 16)]
        for l in range(16 - CEC % 16):
            sv = jnp.full((16,), wv[l + CEC % 16], jnp.float32)
            j = (CEC // 16) * 16 + l
            for d in range(D // 16):
                sl = pl.ds(d * 16, 16)
                msgb[b][j, sl] = rowsb[b][j, sl] * sv

    def pair_body(p, _):
        for b in range(2):
            g = 2 * p + b
            nb = 1 - b
            if b == 0:
                load_idx(g + 1, nb)
                pltpu.async_copy(table_h.at[srcb[nb]], rowsb[nb], semG[nb])
            else:
                @pl.when(p < NPC - 1)
                def _():
                    load_idx(g + 1, nb)
                    pltpu.async_copy(table_h.at[srcb[nb]], rowsb[nb],
                                     semG[nb])

            pltpu.make_async_copy(table_h.at[srcb[b]], rowsb[b],
                                  semG[b]).wait()

            @pl.when(p > 0)
            def _():
                pltpu.make_async_copy(msgb[b], acc_sh.at[dstb[b]],
                                      semSc[b]).wait()

            scale_rows(b)
            pltpu.async_copy(msgb[b], acc_sh.at[dstb[b]], semSc[b],
                             add=True)
        return 0

    lax.fori_loop(0, NPC, pair_body, 0)
    for b in range(2):
        pltpu.make_async_copy(msgb[b], acc_sh.at[dstb[b]], semSc[b]).wait()
    _acc_epilogue(acc_sh, out0_h, out1_h, cid, sid)


@functools.partial(
    pl.kernel,
    out_type=(jax.ShapeDtypeStruct((N, D), jnp.float32),
              jax.ShapeDtypeStruct((N, D), jnp.float32)),
    mesh=_mesh,
    scratch_types=[
        pltpu.VMEM((CE,), jnp.int32),
        pltpu.VMEM((CE,), jnp.float32),
        pltpu.VMEM((CE, D), jnp.float32),
        pltpu.VMEM_SHARED((N, D), jnp.float32),
        pltpu.SemaphoreType.DMA,
    ],
)
def _sc_deg(dst_h, w_h, z_h, out0_h, out1_h, dst_v, w_v, rows_v, acc_sh, sem):
    cid = lax.axis_index("c")
    sid = lax.axis_index("s")
    wid = sid * NC + cid
    _acc_prologue(z_h, acc_sh, sid)

    def chunk_body(ci, _):
        base = wid * EPW + ci * CE
        pltpu.sync_copy(dst_h.at[pl.ds(base, CE)], dst_v)
        pltpu.sync_copy(w_h.at[pl.ds(base, CE)], w_v)

        def group_body(g, _):
            wv = w_v[pl.ds(g * 16, 16)]
            for l in range(16):
                sv = jnp.full((16,), wv[l], jnp.float32)
                j = g * 16 + l
                for d in range(D // 16):
                    rows_v[j, pl.ds(d * 16, 16)] = sv
            return 0

        lax.fori_loop(0, CE // 16, group_body, 0)
        pltpu.sync_copy(rows_v, acc_sh.at[dst_v], add=True)
        return 0

    lax.fori_loop(0, NCHUNK, chunk_body, 0)
    _acc_epilogue(acc_sh, out0_h, out1_h, cid, sid)


CES = 40
NCHS = EPW // CES
NPS = NCHS // 2


@functools.partial(
    pl.kernel,
    out_type=jax.ShapeDtypeStruct((E * 16,), jnp.float32),
    mesh=_mesh,
    scratch_types=(
        [pltpu.VMEM((CES,), jnp.int32)] * 4
        + [pltpu.VMEM((CES, HI), jnp.float32)] * 4
        + [pltpu.VMEM((HI,), jnp.float32)]
        + [pltpu.VMEM((CES * 16,), jnp.float32)] * 2
        + [pltpu.SemaphoreType.DMA] * 6
    ),
)
def _sc_score(a_h, b_h, src_h, dst_h, wm2_h, s16_h,
              src0, src1, dst0, dst1, a0, a1, b0, b1, w2_v, s0, s1,
              semA0, semA1, semB0, semB1, semS0, semS1):
    cid = lax.axis_index("c")
    sid = lax.axis_index("s")
    wid = sid * NC + cid
    ebase = wid * EPW
    srcb, dstb = (src0, src1), (dst0, dst1)
    ab, bb, sb = (a0, a1), (b0, b1), (s0, s1)
    semA, semB, semS = (semA0, semA1), (semB0, semB1), (semS0, semS1)
    pltpu.sync_copy(wm2_h, w2_v)
    w2regs = [_bf16r(w2_v[pl.ds(d * 16, 16)]) for d in range(HI // 16)]

    def load_idx(g, nb):
        base = ebase + g * CES
        pltpu.sync_copy(src_h.at[pl.ds(base, CES)], srcb[nb])
        pltpu.sync_copy(dst_h.at[pl.ds(base, CES)], dstb[nb])

    def issue_gather(nb):
        pltpu.async_copy(a_h.at[srcb[nb]], ab[nb], semA[nb])
        pltpu.async_copy(b_h.at[dstb[nb]], bb[nb], semB[nb])

    load_idx(0, 0)
    issue_gather(0)

    def pair_body(p, _):
        for b in range(2):
            g = 2 * p + b
            nb = 1 - b
            if b == 0:
                load_idx(g + 1, nb)
                issue_gather(nb)
            else:
                @pl.when(p < NPS - 1)
                def _():
                    load_idx(g + 1, nb)
                    issue_gather(nb)

            pltpu.make_async_copy(a_h.at[srcb[b]], ab[b], semA[b]).wait()
            pltpu.make_async_copy(b_h.at[dstb[b]], bb[b], semB[b]).wait()

            @pl.when(p > 0)
            def _():
                pltpu.make_async_copy(
                    sb[b], s16_h.at[pl.ds(0, CES * 16)], semS[b]).wait()

            def edge_body(j, _):
                acc = jnp.zeros((16,), jnp.float32)
                for d in range(HI // 16):
                    sl = pl.ds(d * 16, 16)
                    hid = _bf16r(jnp.maximum(ab[b][j, sl] + bb[b][j, sl],
                                             0.0))
                    acc = acc + hid * w2regs[d]
                sb[b][pl.ds(j * 16, 16)] = acc
                return 0

            lax.fori_loop(0, CES, edge_body, 0)
            pltpu.async_copy(
                sb[b], s16_h.at[pl.ds((ebase + g * CES) * 16, CES * 16)],
                semS[b])
        return 0

    lax.fori_loop(0, NPS, pair_body, 0)
    for b in range(2):
        pltpu.make_async_copy(
            sb[b], s16_h.at[pl.ds(0, CES * 16)], semS[b]).wait()


def _rowspec(w):
    return pl.BlockSpec((RB, w), lambda i: (i, 0))


def _full(shape):
    return pl.BlockSpec(shape, lambda i: tuple(0 for _ in shape))


def _tc_call(body, n_out, out_w, grid=N // RB):
    outs = tuple(jax.ShapeDtypeStruct((grid * RB, w), jnp.float32)
                 for w in out_w)
    return functools.partial(
        pl.pallas_call, body, grid=(grid,),
        out_shape=outs if n_out > 1 else outs[0],
        out_specs=tuple(_rowspec(w) for w in out_w) if n_out > 1
        else _rowspec(out_w[0]))


def _tca_body(d0, d1, x, w1, v1, dinv16, hs1, xv1):
    dinv = 1.0 / jnp.sqrt(jnp.maximum(d0[:, 0:1] + d1[:, 0:1], 1e-12))
    dinv16[...] = jnp.broadcast_to(dinv, (RB, 16))
    hs1[...] = jnp.dot(x[...], w1[...],
                       preferred_element_type=jnp.float32) * dinv
    xv1[...] = jnp.dot(x[...], v1[...], preferred_element_type=jnp.float32)


def _tcb_body(a0, a1, dinv16, xv1, w2, v2, hs2, xv2):
    dinv = dinv16[:, 0:1]
    x1 = jnp.maximum((a0[...] + a1[...]) * dinv + xv1[...], 0.0)
    hs2[...] = jnp.dot(x1, w2[...], preferred_element_type=jnp.float32) * dinv
    xv2[...] = jnp.dot(x1, v2[...], preferred_element_type=jnp.float32)


def _tcc_body(a0, a1, dinv16, xv2, wm1a, wm1b, bm1, x2o, ao, bo):
    x2 = (a0[...] + a1[...]) * dinv16[:, 0:1] + xv2[...]
    x2o[...] = x2
    ao[...] = jnp.dot(x2, wm1a[...], preferred_element_type=jnp.float32)
    bo[...] = jnp.dot(x2, wm1b[...], preferred_element_type=jnp.float32) + bm1[...]


def _tcd1_body(s16r, bm2, so):
    ii = lax.broadcasted_iota(jnp.int32, (128, 8), 0)
    jj = lax.broadcasted_iota(jnp.int32, (128, 8), 1)
    bd = ((ii // 16) == jj).astype(jnp.float32)
    so[...] = jnp.dot(s16r[...], bd, preferred_element_type=jnp.float32,
                      precision=lax.Precision.HIGHEST) + bm2[...]


def _tcd2_body(s2d, ew2d, wo):
    s = s2d[...]
    bits = lax.bitcast_convert_type(s, jnp.int32)
    key = jnp.where(bits >= 0, bits, bits ^ jnp.int32(0x7FFFFFFF))

    def bisect(i, P):
        cand = P | (jnp.int32(1) << (31 - i))
        c = jnp.sum((key >= (cand ^ MIN32)).astype(jnp.int32))
        return jnp.where(c >= K, cand, P)

    T = lax.fori_loop(0, 32, bisect, jnp.int32(0)) ^ MIN32
    gt = key > T
    need = jnp.int32(K) - jnp.sum(gt.astype(jnp.int32))
    eq = key == T
    sh = s.shape
    lin = (lax.broadcasted_iota(jnp.int32, sh, 0) * sh[1]
           + lax.broadcasted_iota(jnp.int32, sh, 1))

    def tie_body(c):
        last, rem = c
        m = jnp.min(jnp.where(eq & (lin > last), lin, jnp.int32(2147483647)))
        return m, rem - 1

    last, _ = lax.while_loop(lambda c: c[1] > 0, tie_body,
                             (jnp.int32(-1), need))
    sel = gt | (eq & (lin <= last))
    wo[...] = jnp.where(sel, ew2d[...] * jax.nn.sigmoid(s), 0.0)


def _tce_body(d0, d1, x2, we1, ve1, dinv16, hs3, xv3):
    dinv = 1.0 / jnp.sqrt(jnp.maximum(d0[:, 0:1] + d1[:, 0:1], 1e-12))
    dinv16[...] = jnp.broadcast_to(dinv, (RB, 16))
    hs3[...] = jnp.dot(x2[...], we1[...],
                       preferred_element_type=jnp.float32) * dinv
    xv3[...] = jnp.dot(x2[...], ve1[...], preferred_element_type=jnp.float32)


def _tcf_body(a0, a1, dinv16, xv3, we2, ve2, h4so, xv4o):
    dinv = dinv16[:, 0:1]
    node3 = jnp.maximum((a0[...] + a1[...]) * dinv + xv3[...], 0.0)
    h4so[...] = jnp.dot(node3, we2[...],
                        preferred_element_type=jnp.float32) * dinv
    xv4o[...] = jnp.dot(node3, ve2[...], preferred_element_type=jnp.float32)


def _tcg_body(z0, z1, dinv16, xv4, wf, batch, bf, out):
    node4 = (z0[...] + z1[...]) * dinv16[:, 0:1] + xv4[...]
    oh = (batch[...] == lax.broadcasted_iota(jnp.int32, (G, 1), 0)
          ).astype(jnp.float32)
    counts = jnp.sum(oh, axis=1, keepdims=True)
    pooled = jnp.dot(oh, node4, preferred_element_type=jnp.float32,
                     precision=lax.Precision.HIGHEST)
    gq = pooled / jnp.maximum(counts, 1.0)
    out[...] = jnp.dot(gq, wf[...],
                       preferred_element_type=jnp.float32) + bf[...]


def kernel(x, edge_index, edge_attr, batch, W1a, V1a, W2a, V2a, Wm1, bm1,
           Wm2, bm2, We1, Ve1, We2, Ve2, Wh1, bh1, Wh2, bh2, Wf, bf):
    f32 = jnp.float32
    src = edge_index[0]
    dst = edge_index[1]
    ew = edge_attr.reshape(E)
    z128 = jnp.zeros((RPT, D), f32)

    d1p0, d1p1 = _sc_deg(dst, ew, z128)

    tca = _tc_call(_tca_body, 3, (16, 128, 128))(
        in_specs=[_rowspec(128), _rowspec(128), _rowspec(128),
                  _full((128, 128)), _full((128, 128))])
    dinv116, hs1, xv1 = tca(d1p0, d1p1, x, W1a, V1a)

    a1p0, a1p1 = _sc_conv(hs1, src, dst, ew, z128)

    tcb = _tc_call(_tcb_body, 2, (128, 128))(
        in_specs=[_rowspec(128), _rowspec(128), _rowspec(16), _rowspec(128),
                  _full((128, 128)), _full((128, 128))])
    hs2, xv2 = tcb(a1p0, a1p1, dinv116, xv1, W2a, V2a)

    a2p0, a2p1 = _sc_conv(hs2, src, dst, ew, z128)

    tcc = _tc_call(_tcc_body, 3, (128, HI, HI))(
        in_specs=[_rowspec(128), _rowspec(128), _rowspec(16), _rowspec(128),
                  _full((128, HI)), _full((128, HI)), _full((1, HI))])
    x2, A, B = tcc(a2p0, a2p1, dinv116, xv2, Wm1[:D], Wm1[D:],
                   bm1.reshape(1, HI))

    s16f = _sc_score(A, B, src, dst, Wm2.reshape(HI))
    s16r = s16f.reshape(E // 8, 128)

    tcd1 = functools.partial(
        pl.pallas_call, _tcd1_body, grid=(E // 8 // 8000,),
        out_shape=jax.ShapeDtypeStruct((E // 8, 8), f32),
        out_specs=pl.BlockSpec((8000, 8), lambda i: (i, 0)))(
            in_specs=[pl.BlockSpec((8000, 128), lambda i: (i, 0)),
                      _full((1, 1))])
    s1 = tcd1(s16r, bm2.reshape(1, 1))
    s2d = s1.reshape(E // 128, 128)

    tcd2 = functools.partial(
        pl.pallas_call, _tcd2_body, grid=(1,),
        out_shape=jax.ShapeDtypeStruct((E // 128, 128), f32),
        out_specs=_full((E // 128, 128)))(
            in_specs=[_full((E // 128, 128)), _full((E // 128, 128))])
    w = tcd2(s2d, ew.reshape(E // 128, 128)).reshape(E)

    d3p0, d3p1 = _sc_deg(dst, w, z128)

    tce = _tc_call(_tce_body, 3, (16, 128, 128))(
        in_specs=[_rowspec(128), _rowspec(128), _rowspec(128),
                  _full((128, 128)), _full((128, 128))])
    dinv316, hs3, xv3 = tce(d3p0, d3p1, x2, We1, Ve1)

    a3p0, a3p1 = _sc_conv(hs3, src, dst, w, z128)

    tcf = _tc_call(_tcf_body, 2, (128, 128))(
        in_specs=[_rowspec(128), _rowspec(128), _rowspec(16), _rowspec(128),
                  _full((128, 128)), _full((128, 128))])
    h4s, xv4 = tcf(a3p0, a3p1, dinv316, xv3, We2, Ve2)

    azp0, azp1 = _sc_conv(h4s, src, dst, w, z128)

    tcg = functools.partial(
        pl.pallas_call, _tcg_body, grid=(1,),
        out_shape=jax.ShapeDtypeStruct((G, 2), f32),
        out_specs=_full((G, 2)))(
            in_specs=[pl.BlockSpec((N, 128), lambda i: (0, 0)),
                      pl.BlockSpec((N, 128), lambda i: (0, 0)),
                      pl.BlockSpec((N, 16), lambda i: (0, 0)),
                      pl.BlockSpec((N, 128), lambda i: (0, 0)),
                      _full((128, 2)),
                      pl.BlockSpec((1, N), lambda i: (0, 0)),
                      _full((1, 2))])
    return tcg(azp0, azp1, dinv316, xv4, Wf,
               batch.reshape(1, N).astype(jnp.int32), bf.reshape(1, 2))


# speedup vs baseline: 5.1143x; 1.4347x over previous
import functools

import jax
import jax.numpy as jnp
import numpy as np
from jax import lax
from jax.experimental import pallas as pl
from jax.experimental.pallas import tpu as pltpu
from jax.experimental.pallas import tpu_sc as plsc

N = 10000
E = 320000
D = 128
G = 128
K = E // 2

NC = 2
NS = 16
NW = NC * NS
EPW = E // NW
CE = 80
NCHUNK = EPW // CE
RPT = 624
RB = 1000
HI = 512
MIN32 = np.int32(-2147483648)
MIN32XF = np.int32(-65536)

_mesh = plsc.VectorSubcoreMesh(core_axis_name="c", subcore_axis_name="s")


def _bf16r(x):
    u = lax.bitcast_convert_type(x, jnp.int32)
    r = (u + jnp.int32(0x7FFF) + ((u >> 16) & 1)) & MIN32XF
    return lax.bitcast_convert_type(r, jnp.float32)


def _acc_prologue(z_h, acc_sh, sid):
    pltpu.sync_copy(z_h.at[pl.ds(0, RPT)], acc_sh.at[pl.ds(sid * RPT, RPT)])

    @pl.when(sid == NS - 1)
    def _():
        pltpu.sync_copy(z_h.at[pl.ds(0, N - NS * RPT)],
                        acc_sh.at[pl.ds(NS * RPT, N - NS * RPT)])

    plsc.subcore_barrier()


def _acc_epilogue(acc_sh, out0_h, out1_h, cid, sid):
    plsc.subcore_barrier()
    rs = pl.ds(sid * RPT, RPT)
    tail = pl.ds(NS * RPT, N - NS * RPT)

    @pl.when(cid == 0)
    def _():
        pltpu.sync_copy(acc_sh.at[rs], out0_h.at[rs])

        @pl.when(sid == NS - 1)
        def _():
            pltpu.sync_copy(acc_sh.at[tail], out0_h.at[tail])

    @pl.when(cid == 1)
    def _():
        pltpu.sync_copy(acc_sh.at[rs], out1_h.at[rs])

        @pl.when(sid == NS - 1)
        def _():
            pltpu.sync_copy(acc_sh.at[tail], out1_h.at[tail])


CEC = 40
NCHC = EPW // CEC
NPC = NCHC // 2


@functools.partial(
    pl.kernel,
    out_type=(jax.ShapeDtypeStruct((N, D), jnp.float32),
              jax.ShapeDtypeStruct((N, D), jnp.float32)),
    mesh=_mesh,
    scratch_types=(
        [pltpu.VMEM((EPW,), jnp.int32)] * 2
        + [pltpu.VMEM((EPW,), jnp.float32)]
        + [pltpu.VMEM((CEC,), jnp.int32)] * 2
        + [pltpu.VMEM((CEC, D), jnp.float32)] * 3
        + [pltpu.VMEM_SHARED((N, D), jnp.float32)]
        + [pltpu.SemaphoreType.DMA] * 2
    ),
)
def _sc_conv(table_h, src_h, dst_h, w_h, z_h, out0_h, out1_h,
             srcall, dstall, wall, dst0, dst1, rows0, rows1, msg_v,
             acc_sh, semG0, semG1):
    cid = lax.axis_index("c")
    sid = lax.axis_index("s")
    wid = sid * NC + cid
    ebase = wid * EPW
    dstb = (dst0, dst1)
    rowsb = (rows0, rows1)
    semG = (semG0, semG1)
    pltpu.sync_copy(src_h.at[pl.ds(ebase, EPW)], srcall)
    pltpu.sync_copy(dst_h.at[pl.ds(ebase, EPW)], dstall)
    pltpu.sync_copy(w_h.at[pl.ds(ebase, EPW)], wall)
    _acc_prologue(z_h, acc_sh, sid)

    def stage_dst(g, nb):
        off = g * CEC
        dstb[nb][pl.ds(0, 16)] = dstall[pl.ds(off, 16)]
        dstb[nb][pl.ds(16, 16)] = dstall[pl.ds(off + 16, 16)]
        dstb[nb][pl.ds(CEC - 16, 16)] = dstall[pl.ds(off + CEC - 16, 16)]

    def issue_gather(g, nb):
        pltpu.async_copy(table_h.at[srcall.at[pl.ds(g * CEC, CEC)]],
                         rowsb[nb], semG[nb])

    stage_dst(0, 0)
    issue_gather(0, 0)

    def scale_rows(g, b):
        def mul16(wv, j0, lane0, nl):
            for l in range(nl):
                sv = jnp.full((16,), wv[lane0 + l], jnp.float32)
                j = j0 + l
                for d in range(D // 16):
                    sl = pl.ds(d * 16, 16)
                    msg_v[j, sl] = rowsb[b][j, sl] * sv

        off = g * CEC
        mul16(wall[pl.ds(off, 16)], 0, 0, 16)
        mul16(wall[pl.ds(off + 16, 16)], 16, 0, 16)
        mul16(wall[pl.ds(off + 24, 16)], 32, 8, 8)

    def pair_body(p, _):
        for b in range(2):
            g = 2 * p + b
            nb = 1 - b
            if b == 0:
                stage_dst(g + 1, nb)
                issue_gather(g + 1, nb)
            else:
                @pl.when(p < NPC - 1)
                def _():
                    stage_dst(g + 1, nb)
                    issue_gather(g + 1, nb)

            pltpu.make_async_copy(table_h.at[srcall.at[pl.ds(0, CEC)]],
                                  rowsb[b], semG[b]).wait()
            scale_rows(g, b)
            pltpu.sync_copy(msg_v, acc_sh.at[dstb[b]], add=True)
        return 0

    lax.fori_loop(0, NPC, pair_body, 0)
    _acc_epilogue(acc_sh, out0_h, out1_h, cid, sid)


@functools.partial(
    pl.kernel,
    out_type=(jax.ShapeDtypeStruct((N, D), jnp.float32),
              jax.ShapeDtypeStruct((N, D), jnp.float32)),
    mesh=_mesh,
    scratch_types=(
        [pltpu.VMEM((EPW,), jnp.int32)]
        + [pltpu.VMEM((EPW,), jnp.float32)]
        + [pltpu.VMEM((CEC,), jnp.int32)] * 2
        + [pltpu.VMEM((CEC, D), jnp.float32)]
        + [pltpu.VMEM_SHARED((N, D), jnp.float32)]
    ),
)
def _sc_deg(dst_h, w_h, z_h, out0_h, out1_h,
            dstall, wall, dst0, dst1, msg_v, acc_sh):
    cid = lax.axis_index("c")
    sid = lax.axis_index("s")
    wid = sid * NC + cid
    ebase = wid * EPW
    dstb = (dst0, dst1)
    pltpu.sync_copy(dst_h.at[pl.ds(ebase, EPW)], dstall)
    pltpu.sync_copy(w_h.at[pl.ds(ebase, EPW)], wall)
    _acc_prologue(z_h, acc_sh, sid)

    def stage_dst(g, nb):
        off = g * CEC
        dstb[nb][pl.ds(0, 16)] = dstall[pl.ds(off, 16)]
        dstb[nb][pl.ds(16, 16)] = dstall[pl.ds(off + 16, 16)]
        dstb[nb][pl.ds(CEC - 16, 16)] = dstall[pl.ds(off + CEC - 16, 16)]

    def fill_rows(g):
        def bc16(wv, j0, lane0, nl):
            for l in range(nl):
                sv = jnp.full((16,), wv[lane0 + l], jnp.float32)
                for d in range(D // 16):
                    msg_v[j0 + l, pl.ds(d * 16, 16)] = sv

        off = g * CEC
        bc16(wall[pl.ds(off, 16)], 0, 0, 16)
        bc16(wall[pl.ds(off + 16, 16)], 16, 0, 16)
        bc16(wall[pl.ds(off + 24, 16)], 32, 8, 8)

    stage_dst(0, 0)

    def pair_body(p, _):
        for b in range(2):
            g = 2 * p + b
            nb = 1 - b
            if b == 0:
                stage_dst(g + 1, nb)
            else:
                @pl.when(p < NPC - 1)
                def _():
                    stage_dst(g + 1, nb)
            fill_rows(g)
            pltpu.sync_copy(msg_v, acc_sh.at[dstb[b]], add=True)
        return 0

    lax.fori_loop(0, NPC, pair_body, 0)
    _acc_epilogue(acc_sh, out0_h, out1_h, cid, sid)


CES = 40
NCHS = EPW // CES
NPS = NCHS // 2


@functools.partial(
    pl.kernel,
    out_type=jax.ShapeDtypeStruct((E * 16,), jnp.float32),
    mesh=_mesh,
    scratch_types=(
        [pltpu.VMEM((EPW,), jnp.int32)] * 2
        + [pltpu.VMEM((CES, HI), jnp.float32)] * 4
        + [pltpu.VMEM((HI,), jnp.float32)]
        + [pltpu.VMEM((CES * 16,), jnp.float32)] * 2
        + [pltpu.SemaphoreType.DMA] * 6
    ),
)
def _sc_score(a_h, b_h, src_h, dst_h, wm2_h, s16_h,
              srcall, dstall, a0, a1, b0, b1, w2_v, s0, s1,
              semA0, semA1, semB0, semB1, semS0, semS1):
    cid = lax.axis_index("c")
    sid = lax.axis_index("s")
    wid = sid * NC + cid
    ebase = wid * EPW
    ab, bb, sb = (a0, a1), (b0, b1), (s0, s1)
    semA, semB, semS = (semA0, semA1), (semB0, semB1), (semS0, semS1)
    pltpu.sync_copy(wm2_h, w2_v)
    pltpu.sync_copy(src_h.at[pl.ds(ebase, EPW)], srcall)
    pltpu.sync_copy(dst_h.at[pl.ds(ebase, EPW)], dstall)
    w2regs = [_bf16r(w2_v[pl.ds(d * 16, 16)]) for d in range(HI // 16)]

    def issue_gather(g, nb):
        sl = pl.ds(g * CES, CES)
        pltpu.async_copy(a_h.at[srcall.at[sl]], ab[nb], semA[nb])
        pltpu.async_copy(b_h.at[dstall.at[sl]], bb[nb], semB[nb])

    issue_gather(0, 0)

    def pair_body(p, _):
        for b in range(2):
            g = 2 * p + b
            nb = 1 - b
            if b == 0:
                issue_gather(g + 1, nb)
            else:
                @pl.when(p < NPS - 1)
                def _():
                    issue_gather(g + 1, nb)

            zsl = pl.ds(0, CES)
            pltpu.make_async_copy(a_h.at[srcall.at[zsl]], ab[b],
                                  semA[b]).wait()
            pltpu.make_async_copy(b_h.at[dstall.at[zsl]], bb[b],
                                  semB[b]).wait()

            @pl.when(p > 0)
            def _():
                pltpu.make_async_copy(
                    sb[b], s16_h.at[pl.ds(0, CES * 16)], semS[b]).wait()

            def edge_body(j, _):
                acc = jnp.zeros((16,), jnp.float32)
                for d in range(HI // 16):
                    sl = pl.ds(d * 16, 16)
                    hid = _bf16r(jnp.maximum(ab[b][j, sl] + bb[b][j, sl],
                                             0.0))
                    acc = acc + hid * w2regs[d]
                sb[b][pl.ds(j * 16, 16)] = acc
                return 0

            lax.fori_loop(0, CES, edge_body, 0)
            pltpu.async_copy(
                sb[b], s16_h.at[pl.ds((ebase + g * CES) * 16, CES * 16)],
                semS[b])
        return 0

    lax.fori_loop(0, NPS, pair_body, 0)
    for b in range(2):
        pltpu.make_async_copy(
            sb[b], s16_h.at[pl.ds(0, CES * 16)], semS[b]).wait()


def _rowspec(w):
    return pl.BlockSpec((RB, w), lambda i: (i, 0))


def _full(shape):
    return pl.BlockSpec(shape, lambda i: tuple(0 for _ in shape))


def _tc_call(body, n_out, out_w, grid=N // RB):
    outs = tuple(jax.ShapeDtypeStruct((grid * RB, w), jnp.float32)
                 for w in out_w)
    return functools.partial(
        pl.pallas_call, body, grid=(grid,),
        out_shape=outs if n_out > 1 else outs[0],
        out_specs=tuple(_rowspec(w) for w in out_w) if n_out > 1
        else _rowspec(out_w[0]))


def _tca_body(d0, d1, x, w1, v1, dinv16, hs1, xv1):
    dinv = 1.0 / jnp.sqrt(jnp.maximum(d0[:, 0:1] + d1[:, 0:1], 1e-12))
    dinv16[...] = jnp.broadcast_to(dinv, (RB, 16))
    hs1[...] = jnp.dot(x[...], w1[...],
                       preferred_element_type=jnp.float32) * dinv
    xv1[...] = jnp.dot(x[...], v1[...], preferred_element_type=jnp.float32)


def _tcb_body(a0, a1, dinv16, xv1, w2, v2, hs2, xv2):
    dinv = dinv16[:, 0:1]
    x1 = jnp.maximum((a0[...] + a1[...]) * dinv + xv1[...], 0.0)
    hs2[...] = jnp.dot(x1, w2[...], preferred_element_type=jnp.float32) * dinv
    xv2[...] = jnp.dot(x1, v2[...], preferred_element_type=jnp.float32)


def _tcc_body(a0, a1, dinv16, xv2, wm1a, wm1b, bm1, x2o, ao, bo):
    x2 = (a0[...] + a1[...]) * dinv16[:, 0:1] + xv2[...]
    x2o[...] = x2
    ao[...] = jnp.dot(x2, wm1a[...], preferred_element_type=jnp.float32)
    bo[...] = jnp.dot(x2, wm1b[...], preferred_element_type=jnp.float32) + bm1[...]


def _tcd1_body(s16r, bm2, so):
    ii = lax.broadcasted_iota(jnp.int32, (128, 8), 0)
    jj = lax.broadcasted_iota(jnp.int32, (128, 8), 1)
    bd = ((ii // 16) == jj).astype(jnp.float32)
    so[...] = jnp.dot(s16r[...], bd, preferred_element_type=jnp.float32,
                      precision=lax.Precision.HIGHEST) + bm2[...]


def _tcd2_body(s2d, ew2d, wo):
    s = s2d[...]
    bits = lax.bitcast_convert_type(s, jnp.int32)
    key = jnp.where(bits >= 0, bits, bits ^ jnp.int32(0x7FFFFFFF))

    def bisect(i, P):
        cand = P | (jnp.int32(1) << (31 - i))
        c = jnp.sum((key >= (cand ^ MIN32)).astype(jnp.int32))
        return jnp.where(c >= K, cand, P)

    T = lax.fori_loop(0, 32, bisect, jnp.int32(0)) ^ MIN32
    gt = key > T
    need = jnp.int32(K) - jnp.sum(gt.astype(jnp.int32))
    eq = key == T
    sh = s.shape
    lin = (lax.broadcasted_iota(jnp.int32, sh, 0) * sh[1]
           + lax.broadcasted_iota(jnp.int32, sh, 1))

    def tie_body(c):
        last, rem = c
        m = jnp.min(jnp.where(eq & (lin > last), lin, jnp.int32(2147483647)))
        return m, rem - 1

    last, _ = lax.while_loop(lambda c: c[1] > 0, tie_body,
                             (jnp.int32(-1), need))
    sel = gt | (eq & (lin <= last))
    wo[...] = jnp.where(sel, ew2d[...] * jax.nn.sigmoid(s), 0.0)


def _tce_body(d0, d1, x2, we1, ve1, dinv16, hs3, xv3):
    dinv = 1.0 / jnp.sqrt(jnp.maximum(d0[:, 0:1] + d1[:, 0:1], 1e-12))
    dinv16[...] = jnp.broadcast_to(dinv, (RB, 16))
    hs3[...] = jnp.dot(x2[...], we1[...],
                       preferred_element_type=jnp.float32) * dinv
    xv3[...] = jnp.dot(x2[...], ve1[...], preferred_element_type=jnp.float32)


def _tcf_body(a0, a1, dinv16, xv3, we2, ve2, h4so, xv4o):
    dinv = dinv16[:, 0:1]
    node3 = jnp.maximum((a0[...] + a1[...]) * dinv + xv3[...], 0.0)
    h4so[...] = jnp.dot(node3, we2[...],
                        preferred_element_type=jnp.float32) * dinv
    xv4o[...] = jnp.dot(node3, ve2[...], preferred_element_type=jnp.float32)


def _tcg_body(z0, z1, dinv16, xv4, wf, batch, bf, out):
    node4 = (z0[...] + z1[...]) * dinv16[:, 0:1] + xv4[...]
    oh = (batch[...] == lax.broadcasted_iota(jnp.int32, (G, 1), 0)
          ).astype(jnp.float32)
    counts = jnp.sum(oh, axis=1, keepdims=True)
    pooled = jnp.dot(oh, node4, preferred_element_type=jnp.float32,
                     precision=lax.Precision.HIGHEST)
    gq = pooled / jnp.maximum(counts, 1.0)
    out[...] = jnp.dot(gq, wf[...],
                       preferred_element_type=jnp.float32) + bf[...]


def kernel(x, edge_index, edge_attr, batch, W1a, V1a, W2a, V2a, Wm1, bm1,
           Wm2, bm2, We1, Ve1, We2, Ve2, Wh1, bh1, Wh2, bh2, Wf, bf):
    f32 = jnp.float32
    src = edge_index[0]
    dst = edge_index[1]
    ew = edge_attr.reshape(E)
    z128 = jnp.zeros((RPT, D), f32)

    d1p0, d1p1 = _sc_deg(dst, ew, z128)

    tca = _tc_call(_tca_body, 3, (16, 128, 128))(
        in_specs=[_rowspec(128), _rowspec(128), _rowspec(128),
                  _full((128, 128)), _full((128, 128))])
    dinv116, hs1, xv1 = tca(d1p0, d1p1, x, W1a, V1a)

    a1p0, a1p1 = _sc_conv(hs1, src, dst, ew, z128)

    tcb = _tc_call(_tcb_body, 2, (128, 128))(
        in_specs=[_rowspec(128), _rowspec(128), _rowspec(16), _rowspec(128),
                  _full((128, 128)), _full((128, 128))])
    hs2, xv2 = tcb(a1p0, a1p1, dinv116, xv1, W2a, V2a)

    a2p0, a2p1 = _sc_conv(hs2, src, dst, ew, z128)

    tcc = _tc_call(_tcc_body, 3, (128, HI, HI))(
        in_specs=[_rowspec(128), _rowspec(128), _rowspec(16), _rowspec(128),
                  _full((128, HI)), _full((128, HI)), _full((1, HI))])
    x2, A, B = tcc(a2p0, a2p1, dinv116, xv2, Wm1[:D], Wm1[D:],
                   bm1.reshape(1, HI))

    s16f = _sc_score(A, B, src, dst, Wm2.reshape(HI))
    s16r = s16f.reshape(E // 8, 128)

    tcd1 = functools.partial(
        pl.pallas_call, _tcd1_body, grid=(E // 8 // 8000,),
        out_shape=jax.ShapeDtypeStruct((E // 8, 8), f32),
        out_specs=pl.BlockSpec((8000, 8), lambda i: (i, 0)))(
            in_specs=[pl.BlockSpec((8000, 128), lambda i: (i, 0)),
                      _full((1, 1))])
    s1 = tcd1(s16r, bm2.reshape(1, 1))
    s2d = s1.reshape(E // 128, 128)

    tcd2 = functools.partial(
        pl.pallas_call, _tcd2_body, grid=(1,),
        out_shape=jax.ShapeDtypeStruct((E // 128, 128), f32),
        out_specs=_full((E // 128, 128)))(
            in_specs=[_full((E // 128, 128)), _full((E // 128, 128))])
    w = tcd2(s2d, ew.reshape(E // 128, 128)).reshape(E)

    d3p0, d3p1 = _sc_deg(dst, w, z128)

    tce = _tc_call(_tce_body, 3, (16, 128, 128))(
        in_specs=[_rowspec(128), _rowspec(128), _rowspec(128),
                  _full((128, 128)), _full((128, 128))])
    dinv316, hs3, xv3 = tce(d3p0, d3p1, x2, We1, Ve1)

    a3p0, a3p1 = _sc_conv(hs3, src, dst, w, z128)

    tcf = _tc_call(_tcf_body, 2, (128, 128))(
        in_specs=[_rowspec(128), _rowspec(128), _rowspec(16), _rowspec(128),
                  _full((128, 128)), _full((128, 128))])
    h4s, xv4 = tcf(a3p0, a3p1, dinv316, xv3, We2, Ve2)

    azp0, azp1 = _sc_conv(h4s, src, dst, w, z128)

    tcg = functools.partial(
        pl.pallas_call, _tcg_body, grid=(1,),
        out_shape=jax.ShapeDtypeStruct((G, 2), f32),
        out_specs=_full((G, 2)))(
            in_specs=[pl.BlockSpec((N, 128), lambda i: (0, 0)),
                      pl.BlockSpec((N, 128), lambda i: (0, 0)),
                      pl.BlockSpec((N, 16), lambda i: (0, 0)),
                      pl.BlockSpec((N, 128), lambda i: (0, 0)),
                      _full((128, 2)),
                      pl.BlockSpec((1, N), lambda i: (0, 0)),
                      _full((1, 2))])
    return tcg(azp0, azp1, dinv316, xv4, Wf,
               batch.reshape(1, N).astype(jnp.int32), bf.reshape(1, 2))


# speedup vs baseline: 7.3187x; 1.4310x over previous
import functools

import jax
import jax.numpy as jnp
import numpy as np
from jax import lax
from jax.experimental import pallas as pl
from jax.experimental.pallas import tpu as pltpu
from jax.experimental.pallas import tpu_sc as plsc

N = 10000
E = 320000
D = 128
G = 128
K = E // 2

NC = 2
NS = 16
NW = NC * NS
EPW = E // NW
CE = 80
NCHUNK = EPW // CE
RPT = 624
RB = 1000
HI = 512
MIN32 = np.int32(-2147483648)
MIN32XF = np.int32(-65536)

_mesh = plsc.VectorSubcoreMesh(core_axis_name="c", subcore_axis_name="s")


def _bf16r(x):
    u = lax.bitcast_convert_type(x, jnp.int32)
    r = (u + jnp.int32(0x7FFF) + ((u >> 16) & 1)) & MIN32XF
    return lax.bitcast_convert_type(r, jnp.float32)


def _acc_prologue(z_h, acc_sh, sid):
    pltpu.sync_copy(z_h.at[pl.ds(0, RPT)], acc_sh.at[pl.ds(sid * RPT, RPT)])

    @pl.when(sid == NS - 1)
    def _():
        pltpu.sync_copy(z_h.at[pl.ds(0, N - NS * RPT)],
                        acc_sh.at[pl.ds(NS * RPT, N - NS * RPT)])

    plsc.subcore_barrier()


def _acc_epilogue(acc_sh, out0_h, out1_h, cid, sid):
    plsc.subcore_barrier()
    rs = pl.ds(sid * RPT, RPT)
    tail = pl.ds(NS * RPT, N - NS * RPT)

    @pl.when(cid == 0)
    def _():
        pltpu.sync_copy(acc_sh.at[rs], out0_h.at[rs])

        @pl.when(sid == NS - 1)
        def _():
            pltpu.sync_copy(acc_sh.at[tail], out0_h.at[tail])

    @pl.when(cid == 1)
    def _():
        pltpu.sync_copy(acc_sh.at[rs], out1_h.at[rs])

        @pl.when(sid == NS - 1)
        def _():
            pltpu.sync_copy(acc_sh.at[tail], out1_h.at[tail])


CEC = 40
NCHC = EPW // CEC
NPC = NCHC // 2


@functools.partial(
    pl.kernel,
    out_type=(jax.ShapeDtypeStruct((N, D), jnp.float32),
              jax.ShapeDtypeStruct((N, D), jnp.float32)),
    mesh=_mesh,
    scratch_types=(
        [pltpu.VMEM((EPW,), jnp.int32)] * 2
        + [pltpu.VMEM((EPW,), jnp.float32)]
        + [pltpu.VMEM((CEC,), jnp.int32)] * 2
        + [pltpu.VMEM((CEC, D), jnp.float32)] * 3
        + [pltpu.VMEM_SHARED((N, D), jnp.float32)]
        + [pltpu.SemaphoreType.DMA] * 2
    ),
)
def _sc_conv(table_h, src_h, dst_h, w_h, z_h, out0_h, out1_h,
             srcall, dstall, wall, dst0, dst1, rows0, rows1, msg_v,
             acc_sh, semG0, semG1):
    cid = lax.axis_index("c")
    sid = lax.axis_index("s")
    wid = sid * NC + cid
    ebase = wid * EPW
    dstb = (dst0, dst1)
    rowsb = (rows0, rows1)
    semG = (semG0, semG1)
    pltpu.sync_copy(src_h.at[pl.ds(ebase, EPW)], srcall)
    pltpu.sync_copy(dst_h.at[pl.ds(ebase, EPW)], dstall)
    pltpu.sync_copy(w_h.at[pl.ds(ebase, EPW)], wall)
    _acc_prologue(z_h, acc_sh, sid)

    def stage_dst(g, nb):
        off = g * CEC
        dstb[nb][pl.ds(0, 16)] = dstall[pl.ds(off, 16)]
        dstb[nb][pl.ds(16, 16)] = dstall[pl.ds(off + 16, 16)]
        dstb[nb][pl.ds(CEC - 16, 16)] = dstall[pl.ds(off + CEC - 16, 16)]

    def issue_gather(g, nb):
        pltpu.async_copy(table_h.at[srcall.at[pl.ds(g * CEC, CEC)]],
                         rowsb[nb], semG[nb])

    stage_dst(0, 0)
    issue_gather(0, 0)

    def scale_rows(g, b):
        def mul16(wv, j0, lane0, nl):
            for l in range(nl):
                sv = jnp.full((16,), wv[lane0 + l], jnp.float32)
                j = j0 + l
                for d in range(D // 16):
                    sl = pl.ds(d * 16, 16)
                    msg_v[j, sl] = rowsb[b][j, sl] * sv

        off = g * CEC
        mul16(wall[pl.ds(off, 16)], 0, 0, 16)
        mul16(wall[pl.ds(off + 16, 16)], 16, 0, 16)
        mul16(wall[pl.ds(off + 24, 16)], 32, 8, 8)

    def pair_body(p, _):
        for b in range(2):
            g = 2 * p + b
            nb = 1 - b
            if b == 0:
                stage_dst(g + 1, nb)
                issue_gather(g + 1, nb)
            else:
                @pl.when(p < NPC - 1)
                def _():
                    stage_dst(g + 1, nb)
                    issue_gather(g + 1, nb)

            pltpu.make_async_copy(table_h.at[srcall.at[pl.ds(0, CEC)]],
                                  rowsb[b], semG[b]).wait()
            scale_rows(g, b)
            pltpu.sync_copy(msg_v, acc_sh.at[dstb[b]], add=True)
        return 0

    lax.fori_loop(0, NPC, pair_body, 0)
    _acc_epilogue(acc_sh, out0_h, out1_h, cid, sid)


@functools.partial(
    pl.kernel,
    out_type=(jax.ShapeDtypeStruct((N, D), jnp.float32),
              jax.ShapeDtypeStruct((N, D), jnp.float32)),
    mesh=_mesh,
    scratch_types=(
        [pltpu.VMEM((EPW,), jnp.int32)]
        + [pltpu.VMEM((EPW,), jnp.float32)]
        + [pltpu.VMEM((CEC,), jnp.int32)] * 2
        + [pltpu.VMEM((CEC, D), jnp.float32)]
        + [pltpu.VMEM_SHARED((N, D), jnp.float32)]
    ),
)
def _sc_deg(dst_h, w_h, z_h, out0_h, out1_h,
            dstall, wall, dst0, dst1, msg_v, acc_sh):
    cid = lax.axis_index("c")
    sid = lax.axis_index("s")
    wid = sid * NC + cid
    ebase = wid * EPW
    dstb = (dst0, dst1)
    pltpu.sync_copy(dst_h.at[pl.ds(ebase, EPW)], dstall)
    pltpu.sync_copy(w_h.at[pl.ds(ebase, EPW)], wall)
    _acc_prologue(z_h, acc_sh, sid)

    def stage_dst(g, nb):
        off = g * CEC
        dstb[nb][pl.ds(0, 16)] = dstall[pl.ds(off, 16)]
        dstb[nb][pl.ds(16, 16)] = dstall[pl.ds(off + 16, 16)]
        dstb[nb][pl.ds(CEC - 16, 16)] = dstall[pl.ds(off + CEC - 16, 16)]

    def fill_rows(g):
        def bc16(wv, j0, lane0, nl):
            for l in range(nl):
                sv = jnp.full((16,), wv[lane0 + l], jnp.float32)
                for d in range(D // 16):
                    msg_v[j0 + l, pl.ds(d * 16, 16)] = sv

        off = g * CEC
        bc16(wall[pl.ds(off, 16)], 0, 0, 16)
        bc16(wall[pl.ds(off + 16, 16)], 16, 0, 16)
        bc16(wall[pl.ds(off + 24, 16)], 32, 8, 8)

    stage_dst(0, 0)

    def pair_body(p, _):
        for b in range(2):
            g = 2 * p + b
            nb = 1 - b
            if b == 0:
                stage_dst(g + 1, nb)
            else:
                @pl.when(p < NPC - 1)
                def _():
                    stage_dst(g + 1, nb)
            fill_rows(g)
            pltpu.sync_copy(msg_v, acc_sh.at[dstb[b]], add=True)
        return 0

    lax.fori_loop(0, NPC, pair_body, 0)
    _acc_epilogue(acc_sh, out0_h, out1_h, cid, sid)


CES = 40
NCHS = EPW // CES
NPS = NCHS // 2


@functools.partial(
    pl.kernel,
    out_type=jax.ShapeDtypeStruct((E * 16,), jnp.float32),
    mesh=_mesh,
    scratch_types=(
        [pltpu.VMEM((EPW,), jnp.int32)] * 2
        + [pltpu.VMEM((CES, HI), jnp.float32)] * 4
        + [pltpu.VMEM((HI,), jnp.float32)] * 2
        + [pltpu.VMEM((CES * 16,), jnp.float32)] * 2
        + [pltpu.SemaphoreType.DMA] * 6
    ),
)
def _sc_score(a_h, b_h, src_h, dst_h, wm2_h, s16_h,
              srcall, dstall, a0, a1, b0, b1, w2_v, w2r_v, s0, s1,
              semA0, semA1, semB0, semB1, semS0, semS1):
    cid = lax.axis_index("c")
    sid = lax.axis_index("s")
    wid = sid * NC + cid
    ebase = wid * EPW
    ab, bb, sb = (a0, a1), (b0, b1), (s0, s1)
    semA, semB, semS = (semA0, semA1), (semB0, semB1), (semS0, semS1)
    pltpu.sync_copy(wm2_h, w2_v)
    pltpu.sync_copy(src_h.at[pl.ds(ebase, EPW)], srcall)
    pltpu.sync_copy(dst_h.at[pl.ds(ebase, EPW)], dstall)
    for d in range(HI // 16):
        sl = pl.ds(d * 16, 16)
        w2r_v[sl] = _bf16r(w2_v[sl])

    def issue_gather(g, nb):
        sl = pl.ds(g * CES, CES)
        pltpu.async_copy(a_h.at[srcall.at[sl]], ab[nb], semA[nb])
        pltpu.async_copy(b_h.at[dstall.at[sl]], bb[nb], semB[nb])

    issue_gather(0, 0)

    def pair_body(p, _):
        for b in range(2):
            g = 2 * p + b
            nb = 1 - b
            if b == 0:
                issue_gather(g + 1, nb)
            else:
                @pl.when(p < NPS - 1)
                def _():
                    issue_gather(g + 1, nb)

            zsl = pl.ds(0, CES)
            pltpu.make_async_copy(a_h.at[srcall.at[zsl]], ab[b],
                                  semA[b]).wait()
            pltpu.make_async_copy(b_h.at[dstall.at[zsl]], bb[b],
                                  semB[b]).wait()

            @pl.when(p > 0)
            def _():
                pltpu.make_async_copy(
                    sb[b], s16_h.at[pl.ds(0, CES * 16)], semS[b]).wait()

            def edge_body(j, _):
                accs = [jnp.zeros((16,), jnp.float32) for _ in range(4)]
                for d in range(HI // 16):
                    sl = pl.ds(d * 16, 16)
                    hid = _bf16r(jnp.maximum(ab[b][j, sl] + bb[b][j, sl],
                                             0.0))
                    accs[d % 4] = accs[d % 4] + hid * w2r_v[sl]
                sb[b][pl.ds(j * 16, 16)] = ((accs[0] + accs[1])
                                            + (accs[2] + accs[3]))
                return 0

            lax.fori_loop(0, CES, edge_body, 0)
            pltpu.async_copy(
                sb[b], s16_h.at[pl.ds((ebase + g * CES) * 16, CES * 16)],
                semS[b])
        return 0

    lax.fori_loop(0, NPS, pair_body, 0)
    for b in range(2):
        pltpu.make_async_copy(
            sb[b], s16_h.at[pl.ds(0, CES * 16)], semS[b]).wait()


def _rowspec(w):
    return pl.BlockSpec((RB, w), lambda i: (i, 0))


def _full(shape):
    return pl.BlockSpec(shape, lambda i: tuple(0 for _ in shape))


def _tc_call(body, n_out, out_w, grid=N // RB):
    outs = tuple(jax.ShapeDtypeStruct((grid * RB, w), jnp.float32)
                 for w in out_w)
    return functools.partial(
        pl.pallas_call, body, grid=(grid,),
        out_shape=outs if n_out > 1 else outs[0],
        out_specs=tuple(_rowspec(w) for w in out_w) if n_out > 1
        else _rowspec(out_w[0]))


def _tca_body(d0, d1, x, w1, v1, dinv16, hs1, xv1):
    dinv = 1.0 / jnp.sqrt(jnp.maximum(d0[:, 0:1] + d1[:, 0:1], 1e-12))
    dinv16[...] = jnp.broadcast_to(dinv, (RB, 16))
    hs1[...] = jnp.dot(x[...], w1[...],
                       preferred_element_type=jnp.float32) * dinv
    xv1[...] = jnp.dot(x[...], v1[...], preferred_element_type=jnp.float32)


def _tcb_body(a0, a1, dinv16, xv1, w2, v2, hs2, xv2):
    dinv = dinv16[:, 0:1]
    x1 = jnp.maximum((a0[...] + a1[...]) * dinv + xv1[...], 0.0)
    hs2[...] = jnp.dot(x1, w2[...], preferred_element_type=jnp.float32) * dinv
    xv2[...] = jnp.dot(x1, v2[...], preferred_element_type=jnp.float32)


def _tcc_body(a0, a1, dinv16, xv2, wm1a, wm1b, bm1, x2o, ao, bo):
    x2 = (a0[...] + a1[...]) * dinv16[:, 0:1] + xv2[...]
    x2o[...] = x2
    ao[...] = jnp.dot(x2, wm1a[...], preferred_element_type=jnp.float32)
    bo[...] = jnp.dot(x2, wm1b[...], preferred_element_type=jnp.float32) + bm1[...]


def _tcd1_body(s16r, bm2, so):
    ii = lax.broadcasted_iota(jnp.int32, (128, 8), 0)
    jj = lax.broadcasted_iota(jnp.int32, (128, 8), 1)
    bd = ((ii // 16) == jj).astype(jnp.float32)
    so[...] = jnp.dot(s16r[...], bd, preferred_element_type=jnp.float32,
                      precision=lax.Precision.HIGHEST) + bm2[...]


def _tcd2_body(s2d, ew2d, wo):
    s = s2d[...]
    bits = lax.bitcast_convert_type(s, jnp.int32)
    key = jnp.where(bits >= 0, bits, bits ^ jnp.int32(0x7FFFFFFF))

    def bisect(i, P):
        cand = P | (jnp.int32(1) << (31 - i))
        c = jnp.sum((key >= (cand ^ MIN32)).astype(jnp.int32))
        return jnp.where(c >= K, cand, P)

    T = lax.fori_loop(0, 32, bisect, jnp.int32(0)) ^ MIN32
    gt = key > T
    need = jnp.int32(K) - jnp.sum(gt.astype(jnp.int32))
    eq = key == T
    sh = s.shape
    lin = (lax.broadcasted_iota(jnp.int32, sh, 0) * sh[1]
           + lax.broadcasted_iota(jnp.int32, sh, 1))

    def tie_body(c):
        last, rem = c
        m = jnp.min(jnp.where(eq & (lin > last), lin, jnp.int32(2147483647)))
        return m, rem - 1

    last, _ = lax.while_loop(lambda c: c[1] > 0, tie_body,
                             (jnp.int32(-1), need))
    sel = gt | (eq & (lin <= last))
    wo[...] = jnp.where(sel, ew2d[...] * jax.nn.sigmoid(s), 0.0)


def _tce_body(d0, d1, x2, we1, ve1, dinv16, hs3, xv3):
    dinv = 1.0 / jnp.sqrt(jnp.maximum(d0[:, 0:1] + d1[:, 0:1], 1e-12))
    dinv16[...] = jnp.broadcast_to(dinv, (RB, 16))
    hs3[...] = jnp.dot(x2[...], we1[...],
                       preferred_element_type=jnp.float32) * dinv
    xv3[...] = jnp.dot(x2[...], ve1[...], preferred_element_type=jnp.float32)


def _tcf_body(a0, a1, dinv16, xv3, we2, ve2, h4so, xv4o):
    dinv = dinv16[:, 0:1]
    node3 = jnp.maximum((a0[...] + a1[...]) * dinv + xv3[...], 0.0)
    h4so[...] = jnp.dot(node3, we2[...],
                        preferred_element_type=jnp.float32) * dinv
    xv4o[...] = jnp.dot(node3, ve2[...], preferred_element_type=jnp.float32)


def _tcg_body(z0, z1, dinv16, xv4, wf, batch, bf, out):
    node4 = (z0[...] + z1[...]) * dinv16[:, 0:1] + xv4[...]
    oh = (batch[...] == lax.broadcasted_iota(jnp.int32, (G, 1), 0)
          ).astype(jnp.float32)
    counts = jnp.sum(oh, axis=1, keepdims=True)
    pooled = jnp.dot(oh, node4, preferred_element_type=jnp.float32,
                     precision=lax.Precision.HIGHEST)
    gq = pooled / jnp.maximum(counts, 1.0)
    out[...] = jnp.dot(gq, wf[...],
                       preferred_element_type=jnp.float32) + bf[...]


def kernel(x, edge_index, edge_attr, batch, W1a, V1a, W2a, V2a, Wm1, bm1,
           Wm2, bm2, We1, Ve1, We2, Ve2, Wh1, bh1, Wh2, bh2, Wf, bf):
    f32 = jnp.float32
    src = edge_index[0]
    dst = edge_index[1]
    ew = edge_attr.reshape(E)
    z128 = jnp.zeros((RPT, D), f32)

    d1p0, d1p1 = _sc_deg(dst, ew, z128)

    tca = _tc_call(_tca_body, 3, (16, 128, 128))(
        in_specs=[_rowspec(128), _rowspec(128), _rowspec(128),
                  _full((128, 128)), _full((128, 128))])
    dinv116, hs1, xv1 = tca(d1p0, d1p1, x, W1a, V1a)

    a1p0, a1p1 = _sc_conv(hs1, src, dst, ew, z128)

    tcb = _tc_call(_tcb_body, 2, (128, 128))(
        in_specs=[_rowspec(128), _rowspec(128), _rowspec(16), _rowspec(128),
                  _full((128, 128)), _full((128, 128))])
    hs2, xv2 = tcb(a1p0, a1p1, dinv116, xv1, W2a, V2a)

    a2p0, a2p1 = _sc_conv(hs2, src, dst, ew, z128)

    tcc = _tc_call(_tcc_body, 3, (128, HI, HI))(
        in_specs=[_rowspec(128), _rowspec(128), _rowspec(16), _rowspec(128),
                  _full((128, HI)), _full((128, HI)), _full((1, HI))])
    x2, A, B = tcc(a2p0, a2p1, dinv116, xv2, Wm1[:D], Wm1[D:],
                   bm1.reshape(1, HI))

    s16f = _sc_score(A, B, src, dst, Wm2.reshape(HI))
    s16r = s16f.reshape(E // 8, 128)

    tcd1 = functools.partial(
        pl.pallas_call, _tcd1_body, grid=(E // 8 // 8000,),
        out_shape=jax.ShapeDtypeStruct((E // 8, 8), f32),
        out_specs=pl.BlockSpec((8000, 8), lambda i: (i, 0)))(
            in_specs=[pl.BlockSpec((8000, 128), lambda i: (i, 0)),
                      _full((1, 1))])
    s1 = tcd1(s16r, bm2.reshape(1, 1))
    s2d = s1.reshape(E // 128, 128)

    tcd2 = functools.partial(
        pl.pallas_call, _tcd2_body, grid=(1,),
        out_shape=jax.ShapeDtypeStruct((E // 128, 128), f32),
        out_specs=_full((E // 128, 128)))(
            in_specs=[_full((E // 128, 128)), _full((E // 128, 128))])
    w = tcd2(s2d, ew.reshape(E // 128, 128)).reshape(E)

    d3p0, d3p1 = _sc_deg(dst, w, z128)

    tce = _tc_call(_tce_body, 3, (16, 128, 128))(
        in_specs=[_rowspec(128), _rowspec(128), _rowspec(128),
                  _full((128, 128)), _full((128, 128))])
    dinv316, hs3, xv3 = tce(d3p0, d3p1, x2, We1, Ve1)

    a3p0, a3p1 = _sc_conv(hs3, src, dst, w, z128)

    tcf = _tc_call(_tcf_body, 2, (128, 128))(
        in_specs=[_rowspec(128), _rowspec(128), _rowspec(16), _rowspec(128),
                  _full((128, 128)), _full((128, 128))])
    h4s, xv4 = tcf(a3p0, a3p1, dinv316, xv3, We2, Ve2)

    azp0, azp1 = _sc_conv(h4s, src, dst, w, z128)

    tcg = functools.partial(
        pl.pallas_call, _tcg_body, grid=(1,),
        out_shape=jax.ShapeDtypeStruct((G, 2), f32),
        out_specs=_full((G, 2)))(
            in_specs=[pl.BlockSpec((N, 128), lambda i: (0, 0)),
                      pl.BlockSpec((N, 128), lambda i: (0, 0)),
                      pl.BlockSpec((N, 16), lambda i: (0, 0)),
                      pl.BlockSpec((N, 128), lambda i: (0, 0)),
                      _full((128, 2)),
                      pl.BlockSpec((1, N), lambda i: (0, 0)),
                      _full((1, 2))])
    return tcg(azp0, azp1, dinv316, xv4, Wf,
               batch.reshape(1, N).astype(jnp.int32), bf.reshape(1, 2))
